# Initial kernel scaffold; baseline (speedup 1.0000x reference)
import jax
import jax.numpy as jnp
from jax.experimental import pallas as pl


def kernel(p0, p1, p2, p3, p4, x0, x1, x2, x3, x4, o0, o1, o2, o3, o4, params):
    raise NotImplementedError("write your pallas kernel here")


# speedup vs baseline: 15.1722x; 15.1722x over previous
import functools

import jax
import jax.numpy as jnp
from jax import lax
from jax.experimental import pallas as pl
from jax.experimental.pallas import tpu as pltpu
from jax.experimental.pallas import tpu_sc as plsc

_B = 4
_EPS_BN = 1e-5

_LEVELS = [
    (8192, 2048, 32),
    (2048, 512, 64),
    (512, 128, 128),
    (128, 32, 256),
]
_SC_SPLIT = {
    8192: (32, 256, 128),
    2048: (32, 64, 64),
    512: (32, 16, 16),
    128: (16, 8, 8),
}


def _select_body(pd_ref, ps_ref, idx0_ref, idx1_ref, idx2_ref, w0_ref,
                 w1_ref, w2_ref, *, bd, bs):
    i = pl.program_id(0)
    pd = pd_ref[...]
    ps = ps_ref[0]
    mm = lax.dot_general(pd.astype(jnp.bfloat16), ps.astype(jnp.bfloat16),
                         (((1,), (1,)), ((), ())),
                         preferred_element_type=jnp.float32)
    ps2 = lax.dot_general(jnp.ones((1, 3), jnp.float32), ps * ps,
                          (((1,), (1,)), ((), ())),
                          precision=lax.Precision.HIGHEST,
                          preferred_element_type=jnp.float32)
    d2 = (pd * pd).sum(axis=1, keepdims=True) + ps2 - 2.0 * mm
    iota = lax.broadcasted_iota(jnp.int32, (bd, bs), 1)
    big = jnp.float32(1e30)
    ms, idxs = [], []
    alive = None
    for _ in range(3):
        d2m = d2 if alive is None else jnp.where(alive, d2, big)
        m = jnp.min(d2m, axis=1, keepdims=True)
        idx = jnp.min(jnp.where(d2m <= m, iota, bs), axis=1, keepdims=True)
        ms.append(m)
        idxs.append(idx)
        ni = iota != idx
        alive = ni if alive is None else (alive & ni)
    ws = [1.0 / (jnp.sqrt(jnp.maximum(m, 1e-10)) + 1e-8) for m in ms]
    wsum = ws[0] + ws[1] + ws[2]
    base = i * bs
    for k, (idx_ref, w_ref) in enumerate(
            [(idx0_ref, w0_ref), (idx1_ref, w1_ref), (idx2_ref, w2_ref)]):
        idx_ref[...] = idxs[k] + base
        w_ref[...] = ws[k] / wsum


def _select(p_dst, p_src, n_dst, n_src):
    bd, bs = n_dst // _B, n_src // _B
    ps_b = p_src.reshape(_B, bs, 3)
    col = lambda: pl.BlockSpec((bd, 1), lambda i: (i, 0))
    outs = pl.pallas_call(
        functools.partial(_select_body, bd=bd, bs=bs),
        grid=(_B,),
        in_specs=[
            pl.BlockSpec((bd, 3), lambda i: (i, 0)),
            pl.BlockSpec((1, bs, 3), lambda i: (i, 0, 0)),
        ],
        out_specs=[col(), col(), col(), col(), col(), col()],
        out_shape=[jax.ShapeDtypeStruct((n_dst, 1), jnp.int32)] * 3
        + [jax.ShapeDtypeStruct((n_dst, 1), jnp.float32)] * 3,
    )(p_dst, ps_b)
    return outs[:3], outs[3:]


def _make_sc_gather():
    info = plsc.get_sparse_core_info()
    nc, ns = info.num_cores, info.num_subcores
    mesh = plsc.VectorSubcoreMesh(core_axis_name="c", subcore_axis_name="s")
    out_type = []
    scratch = []
    for n_dst, n_src, c_dst in _LEVELS:
        c_pad = max(2 * c_dst, 128)
        out_type += [jax.ShapeDtypeStruct((n_dst, c_pad), jnp.float32)] * 3
        nw, rpw, cw = _SC_SPLIT[n_dst]
        nbuf = min(3 * (rpw // cw), 3)
        scratch.append(pltpu.VMEM((nbuf, cw, c_pad), jnp.float32))
        scratch.append(pltpu.VMEM((3 * (rpw // cw), cw), jnp.int32))
    scratch += [pltpu.SemaphoreType.DMA] * 6

    @functools.partial(pl.kernel, mesh=mesh, out_type=out_type,
                       scratch_types=scratch)
    def sc_gather(x0s, x1s, x2s, x3s,
                  i00, i01, i02, i10, i11, i12, i20, i21, i22, i30, i31, i32_,
                  g00, g01, g02, g10, g11, g12, g20, g21, g22, g30, g31, g32,
                  buf0, ib0, buf1, ib1, buf2, ib2, buf3, ib3,
                  gsem, wsem, gsem4, wsem4, gsem1, wsem1):
        wid = lax.axis_index("s") * nc + lax.axis_index("c")
        tables = [x0s, x1s, x2s, x3s]
        idxs = [[i00, i01, i02], [i10, i11, i12], [i20, i21, i22],
                [i30, i31, i32_]]
        gs = [[g00, g01, g02], [g10, g11, g12], [g20, g21, g22],
              [g30, g31, g32]]
        bufs = [buf0, buf1, buf2, buf3]
        ibufs = [ib0, ib1, ib2, ib3]

        def units_of(li):
            nw, rpw, cw = _SC_SPLIT[_LEVELS[li][0]]
            return [(k, j, k * (rpw // cw) + j, cw, wid * rpw + j * cw)
                    for k in range(3) for j in range(rpw // cw)]

        def stage_idx(lis):
            for li in lis:
                for k, j, slot, cw, off in units_of(li):
                    pltpu.sync_copy(idxs[li][k].at[pl.ds(off, cw)],
                                    ibufs[li].at[slot])

        def run_levels(lis, gsem_, wsem_):
            copies = []
            for li in lis:
                for k, j, slot, cw, off in units_of(li):
                    copies.append(pltpu.async_copy(
                        tables[li].at[ibufs[li].at[slot]],
                        bufs[li].at[slot], gsem_))
            for cp in copies:
                cp.wait()
            copies = []
            for li in lis:
                for k, j, slot, cw, off in units_of(li):
                    copies.append(pltpu.async_copy(
                        bufs[li].at[slot],
                        gs[li][k].at[pl.ds(off, cw)], wsem_))
            for cp in copies:
                cp.wait()

        stage_idx([0, 1, 2])
        nw4 = _SC_SPLIT[_LEVELS[3][0]][0]

        @pl.when(wid < nw4)
        def _():
            stage_idx([3])
            run_levels([3], gsem4, wsem4)

        run_levels([1, 2], gsem, wsem)
        units1 = units_of(0)
        for wave in range(2):
            wunits = units1[3 * wave:3 * wave + 3]
            copies = []
            for bslot, (k, j, slot, cw, off) in enumerate(wunits):
                copies.append(pltpu.async_copy(
                    tables[0].at[ibufs[0].at[slot]], bufs[0].at[bslot], gsem1))
            for cp in copies:
                cp.wait()
            copies = []
            for bslot, (k, j, slot, cw, off) in enumerate(wunits):
                copies.append(pltpu.async_copy(
                    bufs[0].at[bslot], gs[0][k].at[pl.ds(off, cw)], wsem1))
            for cp in copies:
                cp.wait()

    return sc_gather


def _finish_body(g0_ref, g1_ref, g2_ref, w0_ref, w1_ref, w2_ref, xd_ref,
                 Wc_ref, bc_ref, gc_ref, betac_ref, Wo_ref, bo_ref, go_ref,
                 betao_ref, out_ref, *, c_src):
    g0 = g0_ref[...][:, :c_src]
    g1 = g1_ref[...][:, :c_src]
    g2 = g2_ref[...][:, :c_src]
    h = w0_ref[...] * g0 + w1_ref[...] * g1 + w2_ref[...] * g2
    z1 = jnp.dot(h.astype(jnp.bfloat16), Wc_ref[...].astype(jnp.bfloat16),
                 preferred_element_type=jnp.float32) + bc_ref[...]
    m1 = jnp.mean(z1, axis=0, keepdims=True)
    v1 = jnp.mean((z1 - m1) * (z1 - m1), axis=0, keepdims=True)
    hb = gc_ref[...] * (z1 - m1) / jnp.sqrt(v1 + _EPS_BN) + betac_ref[...]
    x = xd_ref[...] + jnp.maximum(hb, 0.0)
    z2 = jnp.dot(x.astype(jnp.bfloat16), Wo_ref[...].astype(jnp.bfloat16),
                 preferred_element_type=jnp.float32) + bo_ref[...]
    m2 = jnp.mean(z2, axis=0, keepdims=True)
    v2 = jnp.mean((z2 - m2) * (z2 - m2), axis=0, keepdims=True)
    out_ref[...] = jnp.maximum(
        go_ref[...] * (z2 - m2) / jnp.sqrt(v2 + _EPS_BN) + betao_ref[...], 0.0)


def _finish(gks, wks, x_dst, pr):
    n_dst, c_dst = x_dst.shape
    row = lambda v: v.reshape(1, -1)
    args = (*gks, *wks, x_dst, pr['Wc'], row(pr['bc']), row(pr['gc']),
            row(pr['betac']), pr['Wo'], row(pr['bo']), row(pr['go']),
            row(pr['betao']))
    return pl.pallas_call(
        functools.partial(_finish_body, c_src=2 * c_dst),
        out_shape=jax.ShapeDtypeStruct((n_dst, c_dst), jnp.float32),
    )(*args)


def kernel(p0, p1, p2, p3, p4, x0, x1, x2, x3, x4, o0, o1, o2, o3, o4, params):
    p_dsts = [p0, p1, p2, p3]
    p_srcs = [p1, p2, p3, p4]
    x_dsts = [x0, x1, x2, x3]
    x_srcs = [x1, x2, x3, x4]
    prs = [params['up1'], params['up2'], params['up3'], params['up4']]

    idx_all, w_all = [], []
    for li, (n_dst, n_src, c_dst) in enumerate(_LEVELS):
        idxs, ws = _select(p_dsts[li], p_srcs[li], n_dst, n_src)
        idx_all.append([i.reshape(n_dst) for i in idxs])
        w_all.append(ws)

    sc_gather = _make_sc_gather()
    x1p = jnp.pad(x_srcs[0], ((0, 0), (0, 64)))
    g_flat = sc_gather(x1p, x_srcs[1], x_srcs[2], x_srcs[3],
                       *idx_all[0], *idx_all[1], *idx_all[2], *idx_all[3])
    outs = []
    for li, (n_dst, n_src, c_dst) in enumerate(_LEVELS):
        gks = g_flat[3 * li:3 * li + 3]
        outs.append(_finish(gks, w_all[li], x_dsts[li], prs[li]))
    return (outs[0], outs[1], outs[2], outs[3], x4)


# speedup vs baseline: 20.3967x; 1.3444x over previous
import functools

import jax
import jax.numpy as jnp
from jax import lax
from jax.experimental import pallas as pl
from jax.experimental.pallas import tpu as pltpu
from jax.experimental.pallas import tpu_sc as plsc

_B = 4
_EPS_BN = 1e-5


def _topk3(d2, bd, bs):
    iota = lax.broadcasted_iota(jnp.int32, (bd, bs), 1)
    big = jnp.float32(1e30)
    ms, idxs = [], []
    alive = None
    for _ in range(3):
        d2m = d2 if alive is None else jnp.where(alive, d2, big)
        m = jnp.min(d2m, axis=1, keepdims=True)
        idx = jnp.min(jnp.where(d2m <= m, iota, bs), axis=1, keepdims=True)
        ms.append(m)
        idxs.append(idx)
        ni = iota != idx
        alive = ni if alive is None else (alive & ni)
    ws = [1.0 / (jnp.sqrt(jnp.maximum(m, 1e-10)) + 1e-8) for m in ms]
    return ms, idxs, ws, iota


def _d2_block(pd, ps):
    mm = lax.dot_general(pd.astype(jnp.bfloat16), ps.astype(jnp.bfloat16),
                         (((1,), (1,)), ((), ())),
                         preferred_element_type=jnp.float32)
    ps2 = lax.dot_general(jnp.ones((1, 3), jnp.float32), ps * ps,
                          (((1,), (1,)), ((), ())),
                          precision=lax.Precision.HIGHEST,
                          preferred_element_type=jnp.float32)
    return (pd * pd).sum(axis=1, keepdims=True) + ps2 - 2.0 * mm


def _bn_mlp(h, xd, Wc_ref, bc_ref, gc_ref, betac_ref, Wo_ref, bo_ref,
            go_ref, betao_ref):
    z1 = jnp.dot(h.astype(jnp.bfloat16), Wc_ref[...].astype(jnp.bfloat16),
                 preferred_element_type=jnp.float32) + bc_ref[...]
    m1 = jnp.mean(z1, axis=0, keepdims=True)
    v1 = jnp.mean((z1 - m1) * (z1 - m1), axis=0, keepdims=True)
    hb = gc_ref[...] * (z1 - m1) / jnp.sqrt(v1 + _EPS_BN) + betac_ref[...]
    x = xd + jnp.maximum(hb, 0.0)
    z2 = jnp.dot(x.astype(jnp.bfloat16), Wo_ref[...].astype(jnp.bfloat16),
                 preferred_element_type=jnp.float32) + bo_ref[...]
    m2 = jnp.mean(z2, axis=0, keepdims=True)
    v2 = jnp.mean((z2 - m2) * (z2 - m2), axis=0, keepdims=True)
    return jnp.maximum(
        go_ref[...] * (z2 - m2) / jnp.sqrt(v2 + _EPS_BN) + betao_ref[...], 0.0)


def _fused_body(pd_ref, ps_ref, xd_ref, xs_ref, Wc_ref, bc_ref, gc_ref,
                betac_ref, Wo_ref, bo_ref, go_ref, betao_ref, out_ref,
                z1_ref, *, bd, bs):
    i = pl.program_id(0)
    d2 = _d2_block(pd_ref[...], ps_ref[0])
    ms, idxs, ws, iota = _topk3(d2, bd, bs)
    wsum = ws[0] + ws[1] + ws[2]
    W3 = jnp.where(iota == idxs[0], ws[0] / wsum,
                   jnp.where(iota == idxs[1], ws[1] / wsum,
                             jnp.where(iota == idxs[2], ws[2] / wsum, 0.0)))
    xs = xs_ref[...]
    w_hi = W3.astype(jnp.bfloat16)
    w_lo = (W3 - w_hi.astype(jnp.float32)).astype(jnp.bfloat16)
    x_hi = xs.astype(jnp.bfloat16)
    x_lo = (xs - x_hi.astype(jnp.float32)).astype(jnp.bfloat16)
    dot = lambda a, b: lax.dot_general(
        a, b, (((1,), (0,)), ((), ())), preferred_element_type=jnp.float32)
    h = dot(w_hi, x_hi) + (dot(w_hi, x_lo) + dot(w_lo, x_hi))
    z1 = jnp.dot(h.astype(jnp.bfloat16), Wc_ref[...].astype(jnp.bfloat16),
                 preferred_element_type=jnp.float32) + bc_ref[...]
    z1_ref[pl.ds(i * bd, bd), :] = z1

    @pl.when(i == _B - 1)
    def _finish():
        z1f = z1_ref[...]
        m1 = jnp.mean(z1f, axis=0, keepdims=True)
        v1 = jnp.mean((z1f - m1) * (z1f - m1), axis=0, keepdims=True)
        hb = gc_ref[...] * (z1f - m1) / jnp.sqrt(v1 + _EPS_BN) + betac_ref[...]
        x = xd_ref[...] + jnp.maximum(hb, 0.0)
        z2 = jnp.dot(x.astype(jnp.bfloat16), Wo_ref[...].astype(jnp.bfloat16),
                     preferred_element_type=jnp.float32) + bo_ref[...]
        m2 = jnp.mean(z2, axis=0, keepdims=True)
        v2 = jnp.mean((z2 - m2) * (z2 - m2), axis=0, keepdims=True)
        out_ref[...] = jnp.maximum(
            go_ref[...] * (z2 - m2) / jnp.sqrt(v2 + _EPS_BN)
            + betao_ref[...], 0.0)


def _transition_up_fused(p_dst, x_dst, p_src, x_src, pr):
    n_dst, c_dst = x_dst.shape
    n_src, c_src = x_src.shape
    bd, bs = n_dst // _B, n_src // _B
    ps_b = p_src.reshape(_B, bs, 3)
    row = lambda v: v.reshape(1, -1)
    full = lambda a: pl.BlockSpec(a.shape, lambda i: (0, 0))
    grid_spec = pltpu.PrefetchScalarGridSpec(
        num_scalar_prefetch=0,
        grid=(_B,),
        in_specs=[
            pl.BlockSpec((bd, 3), lambda i: (i, 0)),
            pl.BlockSpec((1, bs, 3), lambda i: (i, 0, 0)),
            full(x_dst),
            pl.BlockSpec((bs, c_src), lambda i: (i, 0)),
            full(pr['Wc']), full(row(pr['bc'])), full(row(pr['gc'])),
            full(row(pr['betac'])), full(pr['Wo']), full(row(pr['bo'])),
            full(row(pr['go'])), full(row(pr['betao'])),
        ],
        out_specs=pl.BlockSpec((n_dst, c_dst), lambda i: (0, 0)),
        scratch_shapes=[pltpu.VMEM((n_dst, c_dst), jnp.float32)],
    )
    return pl.pallas_call(
        functools.partial(_fused_body, bd=bd, bs=bs),
        grid_spec=grid_spec,
        out_shape=jax.ShapeDtypeStruct((n_dst, c_dst), jnp.float32),
    )(p_dst, ps_b, x_dst, x_src, pr['Wc'], row(pr['bc']), row(pr['gc']),
      row(pr['betac']), pr['Wo'], row(pr['bo']), row(pr['go']),
      row(pr['betao']))


_N1, _S1, _C1 = 8192, 2048, 32
_CP = 128
_RPW, _CW = 256, 128


def _select_body(pd_ref, ps_ref, idx0_ref, idx1_ref, idx2_ref, w0_ref,
                 w1_ref, w2_ref, *, bd, bs):
    i = pl.program_id(0)
    d2 = _d2_block(pd_ref[...], ps_ref[0])
    ms, idxs, ws, iota = _topk3(d2, bd, bs)
    wsum = ws[0] + ws[1] + ws[2]
    base = i * bs
    for k, (idx_ref, w_ref) in enumerate(
            [(idx0_ref, w0_ref), (idx1_ref, w1_ref), (idx2_ref, w2_ref)]):
        idx_ref[...] = idxs[k] + base
        w_ref[...] = ws[k] / wsum


def _select(p_dst, p_src, n_dst, n_src):
    bd, bs = n_dst // _B, n_src // _B
    ps_b = p_src.reshape(_B, bs, 3)
    col = lambda: pl.BlockSpec((bd, 1), lambda i: (i, 0))
    outs = pl.pallas_call(
        functools.partial(_select_body, bd=bd, bs=bs),
        grid=(_B,),
        in_specs=[
            pl.BlockSpec((bd, 3), lambda i: (i, 0)),
            pl.BlockSpec((1, bs, 3), lambda i: (i, 0, 0)),
        ],
        out_specs=[col()] * 6,
        out_shape=[jax.ShapeDtypeStruct((n_dst, 1), jnp.int32)] * 3
        + [jax.ShapeDtypeStruct((n_dst, 1), jnp.float32)] * 3,
    )(p_dst, ps_b)
    return outs[:3], outs[3:]


def _make_sc_gather():
    info = plsc.get_sparse_core_info()
    nc = info.num_cores
    mesh = plsc.VectorSubcoreMesh(core_axis_name="c", subcore_axis_name="s")
    nchunk = _RPW // _CW
    nbuf = 3 * nchunk

    @functools.partial(
        pl.kernel, mesh=mesh,
        out_type=[jax.ShapeDtypeStruct((_N1, _CP), jnp.float32)] * 3,
        scratch_types=[
            pltpu.VMEM((nbuf, _CW, _CP), jnp.float32),
            pltpu.VMEM((nbuf, _CW), jnp.int32),
            pltpu.SemaphoreType.DMA,
            pltpu.SemaphoreType.DMA,
            pltpu.SemaphoreType.DMA,
        ],
    )
    def sc_gather(xs, i0, i1, i2, g0, g1, g2, buf, ib, isem, gsem, wsem):
        wid = lax.axis_index("s") * nc + lax.axis_index("c")
        idxs, gs = [i0, i1, i2], [g0, g1, g2]
        units = [(k, j, k * nchunk + j, wid * _RPW + j * _CW)
                 for k in range(3) for j in range(nchunk)]
        stage = [pltpu.async_copy(idxs[k].at[pl.ds(off, _CW)], ib.at[slot],
                                  isem)
                 for k, j, slot, off in units]
        for cp in stage:
            cp.wait()
        gets = [pltpu.async_copy(xs.at[ib.at[slot]], buf.at[slot], gsem)
                for k, j, slot, off in units]
        puts = []
        for cp, (k, j, slot, off) in zip(gets, units):
            cp.wait()
            puts.append(pltpu.async_copy(buf.at[slot],
                                         gs[k].at[pl.ds(off, _CW)], wsem))
        for cp in puts:
            cp.wait()

    return sc_gather


def _finish_body(g0_ref, g1_ref, g2_ref, w0_ref, w1_ref, w2_ref, xd_ref,
                 Wc_ref, bc_ref, gc_ref, betac_ref, Wo_ref, bo_ref, go_ref,
                 betao_ref, out_ref, *, c_src):
    h = (w0_ref[...] * g0_ref[...][:, :c_src]
         + w1_ref[...] * g1_ref[...][:, :c_src]
         + w2_ref[...] * g2_ref[...][:, :c_src])
    out_ref[...] = _bn_mlp(h, xd_ref[...], Wc_ref, bc_ref, gc_ref, betac_ref,
                           Wo_ref, bo_ref, go_ref, betao_ref)


def _finish(gks, wks, x_dst, pr, c_src):
    n_dst, c_dst = x_dst.shape
    row = lambda v: v.reshape(1, -1)
    return pl.pallas_call(
        functools.partial(_finish_body, c_src=c_src),
        out_shape=jax.ShapeDtypeStruct((n_dst, c_dst), jnp.float32),
    )(*gks, *wks, x_dst, pr['Wc'], row(pr['bc']), row(pr['gc']),
      row(pr['betac']), pr['Wo'], row(pr['bo']), row(pr['go']),
      row(pr['betao']))


def kernel(p0, p1, p2, p3, p4, x0, x1, x2, x3, x4, o0, o1, o2, o3, o4, params):
    idxs, ws = _select(p0, p1, _N1, _S1)
    sc_gather = _make_sc_gather()
    x1p = jnp.pad(x1, ((0, 0), (0, _CP - x1.shape[1])))
    gks = sc_gather(x1p, *[i.reshape(_N1) for i in idxs])
    x2u = _transition_up_fused(p1, x1, p2, x2, params['up2'])
    x3u = _transition_up_fused(p2, x2, p3, x3, params['up3'])
    x4u = _transition_up_fused(p3, x3, p4, x4, params['up4'])
    x1u = _finish(gks, ws, x0, params['up1'], 2 * _C1)
    return (x1u, x2u, x3u, x4u, x4)


# speedup vs baseline: 21.7403x; 1.0659x over previous
import functools

import jax
import jax.numpy as jnp
from jax import lax
from jax.experimental import pallas as pl
from jax.experimental.pallas import tpu as pltpu
from jax.experimental.pallas import tpu_sc as plsc

_B = 4
_EPS_BN = 1e-5


def _topk3(d2, bd, bs):
    iota = lax.broadcasted_iota(jnp.int32, (bd, bs), 1)
    big = jnp.float32(1e30)
    ms, idxs = [], []
    alive = None
    for _ in range(3):
        d2m = d2 if alive is None else jnp.where(alive, d2, big)
        m = jnp.min(d2m, axis=1, keepdims=True)
        idx = jnp.min(jnp.where(d2m <= m, iota, bs), axis=1, keepdims=True)
        ms.append(m)
        idxs.append(idx)
        ni = iota != idx
        alive = ni if alive is None else (alive & ni)
    ws = [1.0 / (jnp.sqrt(jnp.maximum(m, 1e-10)) + 1e-8) for m in ms]
    return ms, idxs, ws, iota


def _d2_block(pd, ps):
    mm = lax.dot_general(pd.astype(jnp.bfloat16), ps.astype(jnp.bfloat16),
                         (((1,), (1,)), ((), ())),
                         preferred_element_type=jnp.float32)
    ps2 = lax.dot_general(jnp.ones((1, 3), jnp.float32), ps * ps,
                          (((1,), (1,)), ((), ())),
                          precision=lax.Precision.HIGHEST,
                          preferred_element_type=jnp.float32)
    return (pd * pd).sum(axis=1, keepdims=True) + ps2 - 2.0 * mm


def _bn_mlp(h, xd, Wc_ref, bc_ref, gc_ref, betac_ref, Wo_ref, bo_ref,
            go_ref, betao_ref):
    z1 = jnp.dot(h.astype(jnp.bfloat16), Wc_ref[...].astype(jnp.bfloat16),
                 preferred_element_type=jnp.float32) + bc_ref[...]
    m1 = jnp.mean(z1, axis=0, keepdims=True)
    v1 = jnp.mean((z1 - m1) * (z1 - m1), axis=0, keepdims=True)
    hb = gc_ref[...] * (z1 - m1) / jnp.sqrt(v1 + _EPS_BN) + betac_ref[...]
    x = xd + jnp.maximum(hb, 0.0)
    z2 = jnp.dot(x.astype(jnp.bfloat16), Wo_ref[...].astype(jnp.bfloat16),
                 preferred_element_type=jnp.float32) + bo_ref[...]
    m2 = jnp.mean(z2, axis=0, keepdims=True)
    v2 = jnp.mean((z2 - m2) * (z2 - m2), axis=0, keepdims=True)
    return jnp.maximum(
        go_ref[...] * (z2 - m2) / jnp.sqrt(v2 + _EPS_BN) + betao_ref[...], 0.0)


def _fused_body(pd_ref, ps_ref, xd_ref, xs_ref, Wc_ref, bc_ref, gc_ref,
                betac_ref, Wo_ref, bo_ref, go_ref, betao_ref, out_ref,
                z1_ref, *, bd, bs):
    i = pl.program_id(0)
    d2 = _d2_block(pd_ref[...], ps_ref[0])
    ms, idxs, ws, iota = _topk3(d2, bd, bs)
    wsum = ws[0] + ws[1] + ws[2]
    W3 = jnp.where(iota == idxs[0], ws[0] / wsum,
                   jnp.where(iota == idxs[1], ws[1] / wsum,
                             jnp.where(iota == idxs[2], ws[2] / wsum, 0.0)))
    xs = xs_ref[...]
    w_hi = W3.astype(jnp.bfloat16)
    w_lo = (W3 - w_hi.astype(jnp.float32)).astype(jnp.bfloat16)
    x_hi = xs.astype(jnp.bfloat16)
    x_lo = (xs - x_hi.astype(jnp.float32)).astype(jnp.bfloat16)
    dot = lambda a, b: lax.dot_general(
        a, b, (((1,), (0,)), ((), ())), preferred_element_type=jnp.float32)
    h = dot(w_hi, x_hi) + (dot(w_hi, x_lo) + dot(w_lo, x_hi))
    z1 = jnp.dot(h.astype(jnp.bfloat16), Wc_ref[...].astype(jnp.bfloat16),
                 preferred_element_type=jnp.float32) + bc_ref[...]
    z1_ref[pl.ds(i * bd, bd), :] = z1

    @pl.when(i == _B - 1)
    def _finish():
        z1f = z1_ref[...]
        m1 = jnp.mean(z1f, axis=0, keepdims=True)
        v1 = jnp.mean((z1f - m1) * (z1f - m1), axis=0, keepdims=True)
        hb = gc_ref[...] * (z1f - m1) / jnp.sqrt(v1 + _EPS_BN) + betac_ref[...]
        x = xd_ref[...] + jnp.maximum(hb, 0.0)
        z2 = jnp.dot(x.astype(jnp.bfloat16), Wo_ref[...].astype(jnp.bfloat16),
                     preferred_element_type=jnp.float32) + bo_ref[...]
        m2 = jnp.mean(z2, axis=0, keepdims=True)
        v2 = jnp.mean((z2 - m2) * (z2 - m2), axis=0, keepdims=True)
        out_ref[...] = jnp.maximum(
            go_ref[...] * (z2 - m2) / jnp.sqrt(v2 + _EPS_BN)
            + betao_ref[...], 0.0)


def _transition_up_fused(p_dst, x_dst, p_src, x_src, pr):
    n_dst, c_dst = x_dst.shape
    n_src, c_src = x_src.shape
    bd, bs = n_dst // _B, n_src // _B
    ps_b = p_src.reshape(_B, bs, 3)
    row = lambda v: v.reshape(1, -1)
    full = lambda a: pl.BlockSpec(a.shape, lambda i: (0, 0))
    grid_spec = pltpu.PrefetchScalarGridSpec(
        num_scalar_prefetch=0,
        grid=(_B,),
        in_specs=[
            pl.BlockSpec((bd, 3), lambda i: (i, 0)),
            pl.BlockSpec((1, bs, 3), lambda i: (i, 0, 0)),
            full(x_dst),
            pl.BlockSpec((bs, c_src), lambda i: (i, 0)),
            full(pr['Wc']), full(row(pr['bc'])), full(row(pr['gc'])),
            full(row(pr['betac'])), full(pr['Wo']), full(row(pr['bo'])),
            full(row(pr['go'])), full(row(pr['betao'])),
        ],
        out_specs=pl.BlockSpec((n_dst, c_dst), lambda i: (0, 0)),
        scratch_shapes=[pltpu.VMEM((n_dst, c_dst), jnp.float32)],
    )
    return pl.pallas_call(
        functools.partial(_fused_body, bd=bd, bs=bs),
        grid_spec=grid_spec,
        out_shape=jax.ShapeDtypeStruct((n_dst, c_dst), jnp.float32),
    )(p_dst, ps_b, x_dst, x_src, pr['Wc'], row(pr['bc']), row(pr['gc']),
      row(pr['betac']), pr['Wo'], row(pr['bo']), row(pr['go']),
      row(pr['betao']))


_N1, _S1, _C1 = 8192, 2048, 32
_CP = 128
_RPW, _CW = 256, 128


def _select_body(pd_ref, ps_ref, idx0_ref, idx1_ref, idx2_ref, w0_ref,
                 w1_ref, w2_ref, *, bd, bs):
    i = pl.program_id(0)
    pd = pd_ref[...]
    ps = ps_ref[0]
    mm = lax.dot_general(ps.astype(jnp.bfloat16), pd.astype(jnp.bfloat16),
                         (((1,), (1,)), ((), ())),
                         preferred_element_type=jnp.float32)
    pd2 = lax.dot_general(jnp.ones((1, 3), jnp.float32), pd * pd,
                          (((1,), (1,)), ((), ())),
                          precision=lax.Precision.HIGHEST,
                          preferred_element_type=jnp.float32)
    d2 = (ps * ps).sum(axis=1, keepdims=True) + pd2 - 2.0 * mm
    iota = lax.broadcasted_iota(jnp.int32, (bs, bd), 0)
    big = jnp.float32(1e30)
    ms, idxs = [], []
    alive = None
    for _ in range(3):
        d2m = d2 if alive is None else jnp.where(alive, d2, big)
        m = jnp.min(d2m, axis=0, keepdims=True)
        idx = jnp.min(jnp.where(d2m <= m, iota, bs), axis=0, keepdims=True)
        ms.append(m)
        idxs.append(idx)
        ni = iota != idx
        alive = ni if alive is None else (alive & ni)
    ws = [1.0 / (jnp.sqrt(jnp.maximum(m, 1e-10)) + 1e-8) for m in ms]
    wsum = ws[0] + ws[1] + ws[2]
    base = i * bs
    for k, (idx_ref, w_ref) in enumerate(
            [(idx0_ref, w0_ref), (idx1_ref, w1_ref), (idx2_ref, w2_ref)]):
        idx_ref[0] = idxs[k] + base
        w_ref[0] = ws[k] / wsum


def _select(p_dst, p_src, n_dst, n_src):
    bd, bs = n_dst // _B, n_src // _B
    ps_b = p_src.reshape(_B, bs, 3)
    rowspec = lambda: pl.BlockSpec((1, 1, bd), lambda i: (i, 0, 0))
    outs = pl.pallas_call(
        functools.partial(_select_body, bd=bd, bs=bs),
        grid=(_B,),
        in_specs=[
            pl.BlockSpec((bd, 3), lambda i: (i, 0)),
            pl.BlockSpec((1, bs, 3), lambda i: (i, 0, 0)),
        ],
        out_specs=[rowspec()] * 6,
        out_shape=[jax.ShapeDtypeStruct((_B, 1, bd), jnp.int32)] * 3
        + [jax.ShapeDtypeStruct((_B, 1, bd), jnp.float32)] * 3,
    )(p_dst, ps_b)
    return outs[:3], outs[3:]


def _make_sc_gather():
    info = plsc.get_sparse_core_info()
    nc = info.num_cores
    mesh = plsc.VectorSubcoreMesh(core_axis_name="c", subcore_axis_name="s")
    nchunk = _RPW // _CW
    nbuf = 3 * nchunk

    @functools.partial(
        pl.kernel, mesh=mesh,
        out_type=[jax.ShapeDtypeStruct((_N1, _CP), jnp.float32)] * 3,
        scratch_types=[
            pltpu.VMEM((nbuf, _CW, _CP), jnp.float32),
            pltpu.VMEM((nbuf, _CW), jnp.int32),
            pltpu.SemaphoreType.DMA,
            pltpu.SemaphoreType.DMA,
            pltpu.SemaphoreType.DMA,
        ],
    )
    def sc_gather(xs, i0, i1, i2, g0, g1, g2, buf, ib, isem, gsem, wsem):
        wid = lax.axis_index("s") * nc + lax.axis_index("c")
        idxs, gs = [i0, i1, i2], [g0, g1, g2]
        units = [(k, j, k * nchunk + j, wid * _RPW + j * _CW)
                 for k in range(3) for j in range(nchunk)]
        stage = [pltpu.async_copy(idxs[k].at[pl.ds(off, _CW)], ib.at[slot],
                                  isem)
                 for k, j, slot, off in units]
        for cp in stage:
            cp.wait()
        gets = [pltpu.async_copy(xs.at[ib.at[slot]], buf.at[slot], gsem)
                for k, j, slot, off in units]
        puts = []
        for cp, (k, j, slot, off) in zip(gets, units):
            cp.wait()
            puts.append(pltpu.async_copy(buf.at[slot],
                                         gs[k].at[pl.ds(off, _CW)], wsem))
        for cp in puts:
            cp.wait()

    return sc_gather


def _finish_body(g0_ref, g1_ref, g2_ref, w0_ref, w1_ref, w2_ref, xd_ref,
                 Wc_ref, bc_ref, gc_ref, betac_ref, Wo_ref, bo_ref, go_ref,
                 betao_ref, out_ref, *, c_src):
    h = (w0_ref[...] * g0_ref[...][:, :c_src]
         + w1_ref[...] * g1_ref[...][:, :c_src]
         + w2_ref[...] * g2_ref[...][:, :c_src])
    out_ref[...] = _bn_mlp(h, xd_ref[...], Wc_ref, bc_ref, gc_ref, betac_ref,
                           Wo_ref, bo_ref, go_ref, betao_ref)


def _finish(gks, wks, x_dst, pr, c_src):
    n_dst, c_dst = x_dst.shape
    row = lambda v: v.reshape(1, -1)
    return pl.pallas_call(
        functools.partial(_finish_body, c_src=c_src),
        out_shape=jax.ShapeDtypeStruct((n_dst, c_dst), jnp.float32),
    )(*gks, *wks, x_dst, pr['Wc'], row(pr['bc']), row(pr['gc']),
      row(pr['betac']), pr['Wo'], row(pr['bo']), row(pr['go']),
      row(pr['betao']))


def kernel(p0, p1, p2, p3, p4, x0, x1, x2, x3, x4, o0, o1, o2, o3, o4, params):
    idxs, ws = _select(p0, p1, _N1, _S1)
    ws = [w.reshape(_N1, 1) for w in ws]
    sc_gather = _make_sc_gather()
    x1p = jnp.pad(x1, ((0, 0), (0, _CP - x1.shape[1])))
    gks = sc_gather(x1p, *[i.reshape(_N1) for i in idxs])
    x2u = _transition_up_fused(p1, x1, p2, x2, params['up2'])
    x3u = _transition_up_fused(p2, x2, p3, x3, params['up3'])
    x4u = _transition_up_fused(p3, x3, p4, x4, params['up4'])
    x1u = _finish(gks, ws, x0, params['up1'], 2 * _C1)
    return (x1u, x2u, x3u, x4u, x4)


# speedup vs baseline: 21.9396x; 1.0092x over previous
import functools

import jax
import jax.numpy as jnp
from jax import lax
from jax.experimental import pallas as pl
from jax.experimental.pallas import tpu as pltpu
from jax.experimental.pallas import tpu_sc as plsc

_B = 4
_EPS_BN = 1e-5


def _topk3(d2, bd, bs):
    iota = lax.broadcasted_iota(jnp.int32, (bd, bs), 1)
    big = jnp.float32(1e30)
    ms, idxs = [], []
    alive = None
    for _ in range(3):
        d2m = d2 if alive is None else jnp.where(alive, d2, big)
        m = jnp.min(d2m, axis=1, keepdims=True)
        idx = jnp.min(jnp.where(d2m <= m, iota, bs), axis=1, keepdims=True)
        ms.append(m)
        idxs.append(idx)
        ni = iota != idx
        alive = ni if alive is None else (alive & ni)
    ws = [1.0 / (jnp.sqrt(jnp.maximum(m, 1e-10)) + 1e-8) for m in ms]
    return ms, idxs, ws, iota


def _d2_block(pd, ps):
    mm = lax.dot_general(pd.astype(jnp.bfloat16), ps.astype(jnp.bfloat16),
                         (((1,), (1,)), ((), ())),
                         preferred_element_type=jnp.float32)
    ps2 = lax.dot_general(jnp.ones((1, 3), jnp.float32), ps * ps,
                          (((1,), (1,)), ((), ())),
                          precision=lax.Precision.HIGHEST,
                          preferred_element_type=jnp.float32)
    return (pd * pd).sum(axis=1, keepdims=True) + ps2 - 2.0 * mm


def _bn_mlp(h, xd, Wc_ref, bc_ref, gc_ref, betac_ref, Wo_ref, bo_ref,
            go_ref, betao_ref):
    z1 = jnp.dot(h.astype(jnp.bfloat16), Wc_ref[...].astype(jnp.bfloat16),
                 preferred_element_type=jnp.float32) + bc_ref[...]
    m1 = jnp.mean(z1, axis=0, keepdims=True)
    v1 = jnp.mean((z1 - m1) * (z1 - m1), axis=0, keepdims=True)
    hb = gc_ref[...] * (z1 - m1) / jnp.sqrt(v1 + _EPS_BN) + betac_ref[...]
    x = xd + jnp.maximum(hb, 0.0)
    z2 = jnp.dot(x.astype(jnp.bfloat16), Wo_ref[...].astype(jnp.bfloat16),
                 preferred_element_type=jnp.float32) + bo_ref[...]
    m2 = jnp.mean(z2, axis=0, keepdims=True)
    v2 = jnp.mean((z2 - m2) * (z2 - m2), axis=0, keepdims=True)
    return jnp.maximum(
        go_ref[...] * (z2 - m2) / jnp.sqrt(v2 + _EPS_BN) + betao_ref[...], 0.0)


def _fused_body(pd_ref, ps_ref, xd_ref, xs_ref, Wc_ref, bc_ref, gc_ref,
                betac_ref, Wo_ref, bo_ref, go_ref, betao_ref, out_ref,
                z1_ref, *, bd, bs):
    i = pl.program_id(0)
    d2 = _d2_block(pd_ref[...], ps_ref[0])
    ms, idxs, ws, iota = _topk3(d2, bd, bs)
    wsum = ws[0] + ws[1] + ws[2]
    W3 = jnp.where(iota == idxs[0], ws[0] / wsum,
                   jnp.where(iota == idxs[1], ws[1] / wsum,
                             jnp.where(iota == idxs[2], ws[2] / wsum, 0.0)))
    xs = xs_ref[...]
    w_hi = W3.astype(jnp.bfloat16)
    w_lo = (W3 - w_hi.astype(jnp.float32)).astype(jnp.bfloat16)
    x_hi = xs.astype(jnp.bfloat16)
    x_lo = (xs - x_hi.astype(jnp.float32)).astype(jnp.bfloat16)
    dot = lambda a, b: lax.dot_general(
        a, b, (((1,), (0,)), ((), ())), preferred_element_type=jnp.float32)
    h = dot(w_hi, x_hi) + (dot(w_hi, x_lo) + dot(w_lo, x_hi))
    z1 = jnp.dot(h.astype(jnp.bfloat16), Wc_ref[...].astype(jnp.bfloat16),
                 preferred_element_type=jnp.float32) + bc_ref[...]
    z1_ref[pl.ds(i * bd, bd), :] = z1

    @pl.when(i == _B - 1)
    def _finish():
        z1f = z1_ref[...]
        m1 = jnp.mean(z1f, axis=0, keepdims=True)
        v1 = jnp.mean((z1f - m1) * (z1f - m1), axis=0, keepdims=True)
        hb = gc_ref[...] * (z1f - m1) / jnp.sqrt(v1 + _EPS_BN) + betac_ref[...]
        x = xd_ref[...] + jnp.maximum(hb, 0.0)
        z2 = jnp.dot(x.astype(jnp.bfloat16), Wo_ref[...].astype(jnp.bfloat16),
                     preferred_element_type=jnp.float32) + bo_ref[...]
        m2 = jnp.mean(z2, axis=0, keepdims=True)
        v2 = jnp.mean((z2 - m2) * (z2 - m2), axis=0, keepdims=True)
        out_ref[...] = jnp.maximum(
            go_ref[...] * (z2 - m2) / jnp.sqrt(v2 + _EPS_BN)
            + betao_ref[...], 0.0)


def _transition_up_fused(p_dst, x_dst, p_src, x_src, pr):
    n_dst, c_dst = x_dst.shape
    n_src, c_src = x_src.shape
    bd, bs = n_dst // _B, n_src // _B
    ps_b = p_src.reshape(_B, bs, 3)
    row = lambda v: v.reshape(1, -1)
    full = lambda a: pl.BlockSpec(a.shape, lambda i: (0, 0))
    grid_spec = pltpu.PrefetchScalarGridSpec(
        num_scalar_prefetch=0,
        grid=(_B,),
        in_specs=[
            pl.BlockSpec((bd, 3), lambda i: (i, 0)),
            pl.BlockSpec((1, bs, 3), lambda i: (i, 0, 0)),
            full(x_dst),
            pl.BlockSpec((bs, c_src), lambda i: (i, 0)),
            full(pr['Wc']), full(row(pr['bc'])), full(row(pr['gc'])),
            full(row(pr['betac'])), full(pr['Wo']), full(row(pr['bo'])),
            full(row(pr['go'])), full(row(pr['betao'])),
        ],
        out_specs=pl.BlockSpec((n_dst, c_dst), lambda i: (0, 0)),
        scratch_shapes=[pltpu.VMEM((n_dst, c_dst), jnp.float32)],
    )
    return pl.pallas_call(
        functools.partial(_fused_body, bd=bd, bs=bs),
        grid_spec=grid_spec,
        out_shape=jax.ShapeDtypeStruct((n_dst, c_dst), jnp.float32),
    )(p_dst, ps_b, x_dst, x_src, pr['Wc'], row(pr['bc']), row(pr['gc']),
      row(pr['betac']), pr['Wo'], row(pr['bo']), row(pr['go']),
      row(pr['betao']))


_N1, _S1, _C1 = 8192, 2048, 32
_CP = 128
_RPW, _CW = 256, 128


def _select_body(pd_ref, ps_ref, idx0_ref, idx1_ref, idx2_ref, w0_ref,
                 w1_ref, w2_ref, *, bd, bs):
    i = pl.program_id(0)
    pd = pd_ref[...]
    ps = ps_ref[0]
    mm = lax.dot_general(ps.astype(jnp.bfloat16), pd.astype(jnp.bfloat16),
                         (((1,), (1,)), ((), ())),
                         preferred_element_type=jnp.float32)
    pd2 = lax.dot_general(jnp.ones((1, 3), jnp.float32), pd * pd,
                          (((1,), (1,)), ((), ())),
                          precision=lax.Precision.HIGHEST,
                          preferred_element_type=jnp.float32)
    d2 = (ps * ps).sum(axis=1, keepdims=True) + pd2 - 2.0 * mm
    iota = lax.broadcasted_iota(jnp.int32, (bs, bd), 0)
    big = jnp.float32(1e30)
    ms, idxs = [], []
    alive = None
    for _ in range(3):
        d2m = d2 if alive is None else jnp.where(alive, d2, big)
        m = jnp.min(d2m, axis=0, keepdims=True)
        idx = jnp.min(jnp.where(d2m <= m, iota, bs), axis=0, keepdims=True)
        ms.append(m)
        idxs.append(idx)
        ni = iota != idx
        alive = ni if alive is None else (alive & ni)
    ws = [1.0 / (jnp.sqrt(jnp.maximum(m, 1e-10)) + 1e-8) for m in ms]
    wsum = ws[0] + ws[1] + ws[2]
    base = i * bs
    for k, (idx_ref, w_ref) in enumerate(
            [(idx0_ref, w0_ref), (idx1_ref, w1_ref), (idx2_ref, w2_ref)]):
        idx_ref[0] = idxs[k] + base
        w_ref[0] = ws[k] / wsum


def _select(p_dst, p_src, n_dst, n_src):
    bd, bs = n_dst // _B, n_src // _B
    ps_b = p_src.reshape(_B, bs, 3)
    rowspec = lambda: pl.BlockSpec((1, 1, bd), lambda i: (i, 0, 0))
    outs = pl.pallas_call(
        functools.partial(_select_body, bd=bd, bs=bs),
        grid=(_B,),
        in_specs=[
            pl.BlockSpec((bd, 3), lambda i: (i, 0)),
            pl.BlockSpec((1, bs, 3), lambda i: (i, 0, 0)),
        ],
        out_specs=[rowspec()] * 6,
        out_shape=[jax.ShapeDtypeStruct((_B, 1, bd), jnp.int32)] * 3
        + [jax.ShapeDtypeStruct((_B, 1, bd), jnp.float32)] * 3,
    )(p_dst, ps_b)
    return outs[:3], outs[3:]


def _make_sc_gather():
    info = plsc.get_sparse_core_info()
    nc = info.num_cores
    mesh = plsc.VectorSubcoreMesh(core_axis_name="c", subcore_axis_name="s")
    nchunk = _RPW // _CW
    nbuf = 3 * nchunk

    @functools.partial(
        pl.kernel, mesh=mesh,
        out_type=[jax.ShapeDtypeStruct((_N1, _CP), jnp.float32)] * 3,
        scratch_types=[
            pltpu.VMEM((nbuf, _CW, _CP), jnp.float32),
            pltpu.VMEM((nbuf, _CW), jnp.int32),
            pltpu.SemaphoreType.DMA,
            pltpu.SemaphoreType.DMA,
            pltpu.SemaphoreType.DMA,
        ],
    )
    def sc_gather(xs, i0, i1, i2, g0, g1, g2, buf, ib, isem, gsem, wsem):
        wid = lax.axis_index("s") * nc + lax.axis_index("c")
        idxs, gs = [i0, i1, i2], [g0, g1, g2]
        units = [(k, j, k * nchunk + j, wid * _RPW + j * _CW)
                 for k in range(3) for j in range(nchunk)]
        stage = [pltpu.async_copy(idxs[k].at[pl.ds(off, _CW)], ib.at[slot],
                                  isem)
                 for k, j, slot, off in units]
        for cp in stage:
            cp.wait()
        gets = [pltpu.async_copy(xs.at[ib.at[slot]], buf.at[slot], gsem)
                for k, j, slot, off in units]
        puts = []
        for cp, (k, j, slot, off) in zip(gets, units):
            cp.wait()
            puts.append(pltpu.async_copy(buf.at[slot],
                                         gs[k].at[pl.ds(off, _CW)], wsem))
        for cp in puts:
            cp.wait()

    return sc_gather


def _finish_body(g0_ref, g1_ref, g2_ref, w0_ref, w1_ref, w2_ref, xd_ref,
                 Wc_ref, bc_ref, gc_ref, betac_ref, Wo_ref, bo_ref, go_ref,
                 betao_ref, out_ref, z1_ref, *, c_src, bd):
    i = pl.program_id(0)
    h = (w0_ref[...] * g0_ref[...][:, :c_src]
         + w1_ref[...] * g1_ref[...][:, :c_src]
         + w2_ref[...] * g2_ref[...][:, :c_src])
    z1_ref[pl.ds(i * bd, bd), :] = jnp.dot(
        h.astype(jnp.bfloat16), Wc_ref[...].astype(jnp.bfloat16),
        preferred_element_type=jnp.float32) + bc_ref[...]

    @pl.when(i == _B - 1)
    def _():
        z1 = z1_ref[...]
        m1 = jnp.mean(z1, axis=0, keepdims=True)
        v1 = jnp.mean((z1 - m1) * (z1 - m1), axis=0, keepdims=True)
        hb = gc_ref[...] * (z1 - m1) / jnp.sqrt(v1 + _EPS_BN) + betac_ref[...]
        x = xd_ref[...] + jnp.maximum(hb, 0.0)
        z2 = jnp.dot(x.astype(jnp.bfloat16), Wo_ref[...].astype(jnp.bfloat16),
                     preferred_element_type=jnp.float32) + bo_ref[...]
        m2 = jnp.mean(z2, axis=0, keepdims=True)
        v2 = jnp.mean((z2 - m2) * (z2 - m2), axis=0, keepdims=True)
        out_ref[...] = jnp.maximum(
            go_ref[...] * (z2 - m2) / jnp.sqrt(v2 + _EPS_BN)
            + betao_ref[...], 0.0)


def _finish(gks, wks, x_dst, pr, c_src):
    n_dst, c_dst = x_dst.shape
    bd = n_dst // _B
    row = lambda v: v.reshape(1, -1)
    full = lambda a: pl.BlockSpec(a.shape, lambda i: (0, 0))
    gspec = lambda: pl.BlockSpec((bd, _CP), lambda i: (i, 0))
    wspec = lambda: pl.BlockSpec((bd, 1), lambda i: (i, 0))
    return pl.pallas_call(
        functools.partial(_finish_body, c_src=c_src, bd=bd),
        grid=(_B,),
        in_specs=[gspec(), gspec(), gspec(), wspec(), wspec(), wspec(),
                  full(x_dst), full(pr['Wc']), full(row(pr['bc'])),
                  full(row(pr['gc'])), full(row(pr['betac'])),
                  full(pr['Wo']), full(row(pr['bo'])), full(row(pr['go'])),
                  full(row(pr['betao']))],
        out_specs=pl.BlockSpec((n_dst, c_dst), lambda i: (0, 0)),
        out_shape=jax.ShapeDtypeStruct((n_dst, c_dst), jnp.float32),
        scratch_shapes=[pltpu.VMEM((n_dst, c_dst), jnp.float32)],
    )(*gks, *wks, x_dst, pr['Wc'], row(pr['bc']), row(pr['gc']),
      row(pr['betac']), pr['Wo'], row(pr['bo']), row(pr['go']),
      row(pr['betao']))


def kernel(p0, p1, p2, p3, p4, x0, x1, x2, x3, x4, o0, o1, o2, o3, o4, params):
    idxs, ws = _select(p0, p1, _N1, _S1)
    ws = [w.reshape(_N1, 1) for w in ws]
    sc_gather = _make_sc_gather()
    x1p = jnp.pad(x1, ((0, 0), (0, _CP - x1.shape[1])))
    gks = sc_gather(x1p, *[i.reshape(_N1) for i in idxs])
    x2u = _transition_up_fused(p1, x1, p2, x2, params['up2'])
    x3u = _transition_up_fused(p2, x2, p3, x3, params['up3'])
    x4u = _transition_up_fused(p3, x3, p4, x4, params['up4'])
    x1u = _finish(gks, ws, x0, params['up1'], 2 * _C1)
    return (x1u, x2u, x3u, x4u, x4)
